# baseline (speedup 1.0000x reference)
import functools

import jax
import jax.numpy as jnp
from jax import lax
from jax.experimental import pallas as pl
from jax.experimental.pallas import tpu as pltpu
from jax.experimental.pallas import tpu_sc as plsc

N = 10000
E = 320000
D = 128

NC, NS = 2, 16
NW = NC * NS
CH = 128
EPW = -(-E // (NW * CH * 16)) * CH * 16
EP = EPW * NW
NCHUNK = EPW // CH
NPAD = 10240
STR = NPAD // NS

BR = 1000
G = N // BR


PC0 = 40
PC1 = NCHUNK - PC0


def _make_sc_body(with_counts):
    def body(*args):
        if with_counts:
            (sx, srcr, dstr, part, cnt,
             acc_sh, cnt_sh, srcv, dstv, rows, zcnt, ones_v,
             sem0, sem1, sem2, sem3, sem4) = args
        else:
            (sx, srcr, dstr, part, acc_sh, srcv, dstv, rows,
             sem0, sem1, sem3, sem4) = args
        c = lax.axis_index("c")
        s = lax.axis_index("s")
        wid = s * NC + c
        sems = (sem0, sem1)

        zb = rows.at[0]

        def _zrow(r, carry):
            for k in range(D // 16):
                zb[r, pl.ds(k * 16, 16)] = jnp.zeros((16,), jnp.float32)
            return carry

        lax.fori_loop(0, CH, _zrow, 0)
        base = s * STR
        for k in range(STR // CH):
            pltpu.sync_copy(zb, acc_sh.at[pl.ds(base + k * CH, CH)])
        if with_counts:
            def _zc(i, carry):
                zcnt[pl.ds(i * 16, 16)] = jnp.zeros((16,), jnp.float32)
                return carry

            lax.fori_loop(0, STR // 16, _zc, 0)
            for k in range(CH // 16):
                ones_v[pl.ds(k * 16, 16)] = jnp.ones((16,), jnp.float32)
            pltpu.sync_copy(zcnt, cnt_sh.at[pl.ds(base, STR)])
        plsc.subcore_barrier()

        def _wait_g(b):
            pltpu.make_async_copy(sx.at[pl.ds(0, CH)], rows.at[b],
                                  sems[b]).wait()

        def _wait_s(b, sems_s):
            pltpu.make_async_copy(sx.at[pl.ds(0, CH)], rows.at[b],
                                  sems_s[b]).wait()

        for p, pc in enumerate((PC0, PC1)):
            off = p * PC0
            pltpu.sync_copy(srcr.at[wid, pl.ds(off, pc)],
                            srcv.at[pl.ds(0, pc)])
            pltpu.sync_copy(dstr.at[wid, pl.ds(off, pc)],
                            dstv.at[pl.ds(0, pc)])
            pltpu.async_copy(sx.at[srcv.at[0]], rows.at[0], sem0)
            pltpu.async_copy(sx.at[srcv.at[1]], rows.at[1], sem1)
            sems_s = (sem3, sem4)

            def _pair(jj, carry):
                j0 = jj * 2
                j1 = j0 + 1
                _wait_g(0)
                if with_counts:
                    pltpu.async_copy(ones_v, cnt_sh.at[dstv.at[j0]], sem2,
                                     add=True)
                pltpu.async_copy(rows.at[0], acc_sh.at[dstv.at[j0]], sem3,
                                 add=True)
                _wait_g(1)
                if with_counts:
                    pltpu.async_copy(ones_v, cnt_sh.at[dstv.at[j1]], sem2,
                                     add=True)
                pltpu.async_copy(rows.at[1], acc_sh.at[dstv.at[j1]], sem4,
                                 add=True)
                _wait_s(0, sems_s)

                @pl.when(j0 + 2 < pc)
                def _():
                    pltpu.async_copy(sx.at[srcv.at[j0 + 2]], rows.at[0], sem0)
                _wait_s(1, sems_s)

                @pl.when(j1 + 2 < pc)
                def _():
                    pltpu.async_copy(sx.at[srcv.at[j1 + 2]], rows.at[1], sem1)
                return carry

            lax.fori_loop(0, pc // 2, _pair, 0)
            if with_counts:
                pltpu.make_async_copy(srcr.at[wid, pl.ds(0, pc)],
                                      srcv.at[pl.ds(0, pc)], sem2).wait()

        plsc.subcore_barrier()

        for k in range(STR // CH):
            pltpu.sync_copy(acc_sh.at[pl.ds(base + k * CH, CH)],
                            part.at[c, pl.ds(base + k * CH, CH)])
        if with_counts:
            pltpu.sync_copy(cnt_sh.at[pl.ds(base, STR)],
                            cnt.at[c, pl.ds(base, STR)])

    return body


@functools.cache
def _sc_scatter_kernel(with_counts):
    out_type = [jax.ShapeDtypeStruct((NC, NPAD, D), jnp.float32)]
    scratch = [pltpu.VMEM_SHARED((NPAD, D), jnp.float32)]
    if with_counts:
        out_type.append(jax.ShapeDtypeStruct((NC, NPAD), jnp.float32))
        scratch.append(pltpu.VMEM_SHARED((NPAD,), jnp.float32))
    scratch += [
        pltpu.VMEM((PC0, CH), jnp.int32),
        pltpu.VMEM((PC0, CH), jnp.int32),
        pltpu.VMEM((2, CH, D), jnp.float32),
    ]
    if with_counts:
        scratch += [
            pltpu.VMEM((STR,), jnp.float32),
            pltpu.VMEM((CH,), jnp.float32),
        ]
    scratch += [pltpu.SemaphoreType.DMA, pltpu.SemaphoreType.DMA]
    if with_counts:
        scratch.append(pltpu.SemaphoreType.DMA)
    scratch += [pltpu.SemaphoreType.DMA, pltpu.SemaphoreType.DMA]
    return pl.kernel(
        _make_sc_body(with_counts),
        out_type=tuple(out_type),
        mesh=plsc.VectorSubcoreMesh(core_axis_name="c", subcore_axis_name="s",
                                    num_cores=NC, num_subcores=NS),
        scratch_types=tuple(scratch),
    )


def _sc_scatter(sx, src, dst):
    return _sc_scatter_kernel(True)(sx, src, dst)


def _sc_scatter_nc(sx, src, dst):
    return _sc_scatter_kernel(False)(sx, src, dst)[0]


def _lin1_body(xa, xb, wsab, wtab, wsba, wtba, sxab, txab, sxba, txba):
    a = xa[...]
    b = xb[...]
    sxab[...] = jnp.dot(a, wsab[...], preferred_element_type=jnp.float32)
    txab[...] = jnp.dot(b, wtab[...], preferred_element_type=jnp.float32)
    sxba[...] = jnp.dot(b, wsba[...], preferred_element_type=jnp.float32)
    txba[...] = jnp.dot(a, wtba[...], preferred_element_type=jnp.float32)


def _comb_body(pab, cab, txab, pba, cba, txba, wsab, wtab, wsba, wtba,
               sx2ab, tx2ab, sx2ba, tx2ba):
    hb = jnp.maximum(
        txab[...] + (pab[0] + pab[1]) / jnp.maximum(cab[0] + cab[1], 1.0), 0.0)
    ha = jnp.maximum(
        txba[...] + (pba[0] + pba[1]) / jnp.maximum(cba[0] + cba[1], 1.0), 0.0)
    sx2ab[...] = jnp.dot(ha, wsab[...], preferred_element_type=jnp.float32)
    tx2ab[...] = jnp.dot(hb, wtab[...], preferred_element_type=jnp.float32)
    sx2ba[...] = jnp.dot(hb, wsba[...], preferred_element_type=jnp.float32)
    tx2ba[...] = jnp.dot(ha, wtba[...], preferred_element_type=jnp.float32)


def _fin_body(pab, cab, txab, pba, cba, txba, wla, wlb,
              ha_o, hb_o, oa, ob):
    hb = jnp.maximum(
        txab[...] + (pab[0] + pab[1]) / jnp.maximum(cab[0] + cab[1], 1.0), 0.0)
    ha = jnp.maximum(
        txba[...] + (pba[0] + pba[1]) / jnp.maximum(cba[0] + cba[1], 1.0), 0.0)
    ha_o[...] = ha
    hb_o[...] = hb
    oa[...] = jnp.dot(ha, wla[...], preferred_element_type=jnp.float32)
    ob[...] = jnp.dot(hb, wlb[...], preferred_element_type=jnp.float32)


_row_spec = pl.BlockSpec((BR, D), lambda i: (i, 0))
_w_spec = pl.BlockSpec((D, D), lambda i: (0, 0))
_part_spec = pl.BlockSpec((NC, BR, D), lambda i: (0, i, 0))
_cnt_spec = pl.BlockSpec((NC, BR, 1), lambda i: (0, i, 0))

_lin1 = pl.pallas_call(
    _lin1_body,
    grid=(G,),
    in_specs=[_row_spec, _row_spec, _w_spec, _w_spec, _w_spec, _w_spec],
    out_specs=[_row_spec] * 4,
    out_shape=[jax.ShapeDtypeStruct((N, D), jnp.float32)] * 4,
)

_comb = pl.pallas_call(
    _comb_body,
    grid=(G,),
    in_specs=[_part_spec, _cnt_spec, _row_spec,
              _part_spec, _cnt_spec, _row_spec,
              _w_spec, _w_spec, _w_spec, _w_spec],
    out_specs=[_row_spec] * 4,
    out_shape=[jax.ShapeDtypeStruct((N, D), jnp.float32)] * 4,
)

_fin = pl.pallas_call(
    _fin_body,
    grid=(G,),
    in_specs=[_part_spec, _cnt_spec, _row_spec,
              _part_spec, _cnt_spec, _row_spec,
              pl.BlockSpec((D, 1), lambda i: (0, 0)),
              pl.BlockSpec((D, 1), lambda i: (0, 0))],
    out_specs=[_row_spec, _row_spec,
               pl.BlockSpec((BR, 1), lambda i: (i, 0)),
               pl.BlockSpec((BR, 1), lambda i: (i, 0))],
    out_shape=[jax.ShapeDtypeStruct((N, D), jnp.float32),
               jax.ShapeDtypeStruct((N, D), jnp.float32),
               jax.ShapeDtypeStruct((N, 1), jnp.float32),
               jax.ShapeDtypeStruct((N, 1), jnp.float32)],
)


def _prep_edges(edge):
    npad = EP - E
    pad_src = lax.iota(jnp.int32, npad) % N
    pad_dst = N + (lax.iota(jnp.int32, npad) % 16)
    src = jnp.concatenate([edge[0], pad_src]).reshape(NW, NCHUNK, CH)
    dst = jnp.concatenate([edge[1], pad_dst]).reshape(NW, NCHUNK, CH)
    return src, dst


def kernel(x_a, x_b, edge_ab, edge_ba, W_src1_ab, W_tgt1_ab, W_src1_ba,
           W_tgt1_ba, W_src2_ab, W_tgt2_ab, W_src2_ba, W_tgt2_ba,
           W_lin_a, W_lin_b, b_lin_a, b_lin_b):
    src_ab, dst_ab = _prep_edges(edge_ab)
    src_ba, dst_ba = _prep_edges(edge_ba)

    sx1ab, tx1ab, sx1ba, tx1ba = _lin1(
        x_a, x_b, W_src1_ab, W_tgt1_ab, W_src1_ba, W_tgt1_ba)
    pab1, cab = _sc_scatter(sx1ab, src_ab, dst_ab)
    pba1, cba = _sc_scatter(sx1ba, src_ba, dst_ba)
    cab3 = cab.reshape(NC, NPAD, 1)
    cba3 = cba.reshape(NC, NPAD, 1)

    sx2ab, tx2ab, sx2ba, tx2ba = _comb(
        pab1, cab3, tx1ab, pba1, cba3, tx1ba,
        W_src2_ab, W_tgt2_ab, W_src2_ba, W_tgt2_ba)
    pab2 = _sc_scatter_nc(sx2ab, src_ab, dst_ab)
    pba2 = _sc_scatter_nc(sx2ba, src_ba, dst_ba)

    ha, hb, oa, ob = _fin(pab2, cab3, tx2ab, pba2, cba3, tx2ba,
                          W_lin_a, W_lin_b)
    return ha, hb, oa + b_lin_a, ob + b_lin_b


# speedup vs baseline: 1.2863x; 1.2863x over previous
import functools

import jax
import jax.numpy as jnp
from jax import lax
from jax.experimental import pallas as pl
from jax.experimental.pallas import tpu as pltpu
from jax.experimental.pallas import tpu_sc as plsc

N = 10000
E = 320000
D = 128

NC, NS = 2, 16
CH = 128
PC = 40
NPH = 4
NCHUNK = PC * NPH
EPW = NCHUNK * CH
EP = EPW * NS
NPAD = 10240
STR = NPAD // NS

BR = 1000
G = N // BR


def _make_sc_body(with_counts):
    def body(*args):
        if with_counts:
            (sx, srcr, dstr, part, cnt,
             acc_sh, cnt_sh, srcv, dstv, rows, zcnt, ones_v,
             sem0, sem1, sem2) = args
        else:
            (sx, srcr, dstr, part, acc_sh, srcv, dstv, rows, sem0, sem1) = args
        c = lax.axis_index("c")
        s = lax.axis_index("s")
        sems = (sem0, sem1)

        zb = rows.at[0]

        def _zrow(r, carry):
            for k in range(D // 16):
                zb[r, pl.ds(k * 16, 16)] = jnp.zeros((16,), jnp.float32)
            return carry

        lax.fori_loop(0, CH, _zrow, 0)
        base = s * STR
        for k in range(STR // CH):
            pltpu.sync_copy(zb, acc_sh.at[pl.ds(base + k * CH, CH)])
        if with_counts:
            def _zc(i, carry):
                zcnt[pl.ds(i * 16, 16)] = jnp.zeros((16,), jnp.float32)
                return carry

            lax.fori_loop(0, STR // 16, _zc, 0)
            for k in range(CH // 16):
                ones_v[pl.ds(k * 16, 16)] = jnp.ones((16,), jnp.float32)
            pltpu.sync_copy(zcnt, cnt_sh.at[pl.ds(base, STR)])
        plsc.subcore_barrier()

        def _scatter(j, b):
            pltpu.make_async_copy(sx.at[pl.ds(0, CH)], rows.at[b],
                                  sems[b]).wait()
            if with_counts:
                pltpu.async_copy(ones_v, cnt_sh.at[dstv.at[j]], sem2, add=True)
            pltpu.sync_copy(rows.at[b], acc_sh.at[dstv.at[j]], add=True)

        for p in range(NPH):
            off = p * PC
            pltpu.sync_copy(srcr.at[c, s, pl.ds(off, PC)], srcv)
            pltpu.sync_copy(dstr.at[c, s, pl.ds(off, PC)], dstv)
            pltpu.async_copy(sx.at[srcv.at[0]], rows.at[0], sem0)
            pltpu.async_copy(sx.at[srcv.at[1]], rows.at[1], sem1)

            def _pair(jj, carry):
                for b in range(2):
                    j = jj * 2 + b
                    _scatter(j, b)
                    nxt = j + 2

                    @pl.when(nxt < PC)
                    def _():
                        pltpu.async_copy(sx.at[srcv.at[nxt]], rows.at[b],
                                         sems[b])
                return carry

            lax.fori_loop(0, PC // 2, _pair, 0)
            if with_counts:
                pltpu.make_async_copy(srcr.at[c, s, pl.ds(0, PC)],
                                      srcv, sem2).wait()

        plsc.subcore_barrier()

        for k in range(STR // CH):
            pltpu.sync_copy(acc_sh.at[pl.ds(base + k * CH, CH)],
                            part.at[c, pl.ds(base + k * CH, CH)])
        if with_counts:
            pltpu.sync_copy(cnt_sh.at[pl.ds(base, STR)],
                            cnt.at[c, pl.ds(base, STR)])

    return body


@functools.cache
def _sc_scatter_kernel(with_counts):
    out_type = [jax.ShapeDtypeStruct((NC, NPAD, D), jnp.float32)]
    scratch = [pltpu.VMEM_SHARED((NPAD, D), jnp.float32)]
    if with_counts:
        out_type.append(jax.ShapeDtypeStruct((NC, NPAD), jnp.float32))
        scratch.append(pltpu.VMEM_SHARED((NPAD,), jnp.float32))
    scratch += [
        pltpu.VMEM((PC, CH), jnp.int32),
        pltpu.VMEM((PC, CH), jnp.int32),
        pltpu.VMEM((2, CH, D), jnp.float32),
    ]
    if with_counts:
        scratch += [
            pltpu.VMEM((STR,), jnp.float32),
            pltpu.VMEM((CH,), jnp.float32),
        ]
    scratch += [pltpu.SemaphoreType.DMA, pltpu.SemaphoreType.DMA]
    if with_counts:
        scratch.append(pltpu.SemaphoreType.DMA)
    return pl.kernel(
        _make_sc_body(with_counts),
        out_type=tuple(out_type),
        mesh=plsc.VectorSubcoreMesh(core_axis_name="c", subcore_axis_name="s",
                                    num_cores=NC, num_subcores=NS),
        scratch_types=tuple(scratch),
    )


def _sc_scatter(sx, src, dst):
    return _sc_scatter_kernel(True)(sx, src, dst)


def _sc_scatter_nc(sx, src, dst):
    return _sc_scatter_kernel(False)(sx, src, dst)[0]


def _lin1_body(xa, xb, wsab, wtab, wsba, wtba, sxp, txab, txba):
    a = xa[...]
    b = xb[...]
    sxp[0] = jnp.dot(a, wsab[...], preferred_element_type=jnp.float32)
    sxp[1] = jnp.dot(b, wsba[...], preferred_element_type=jnp.float32)
    txab[...] = jnp.dot(b, wtab[...], preferred_element_type=jnp.float32)
    txba[...] = jnp.dot(a, wtba[...], preferred_element_type=jnp.float32)


def _comb_body(part, cnt, txab, txba, wsab, wtab, wsba, wtba,
               sxp2, tx2ab, tx2ba):
    hb = jnp.maximum(
        txab[...] + part[0] / jnp.maximum(cnt[0], 1.0), 0.0)
    ha = jnp.maximum(
        txba[...] + part[1] / jnp.maximum(cnt[1], 1.0), 0.0)
    sxp2[0] = jnp.dot(ha, wsab[...], preferred_element_type=jnp.float32)
    sxp2[1] = jnp.dot(hb, wsba[...], preferred_element_type=jnp.float32)
    tx2ab[...] = jnp.dot(hb, wtab[...], preferred_element_type=jnp.float32)
    tx2ba[...] = jnp.dot(ha, wtba[...], preferred_element_type=jnp.float32)


def _fin_body(part, cnt, txab, txba, wla, wlb, ha_o, hb_o, oa, ob):
    hb = jnp.maximum(
        txab[...] + part[0] / jnp.maximum(cnt[0], 1.0), 0.0)
    ha = jnp.maximum(
        txba[...] + part[1] / jnp.maximum(cnt[1], 1.0), 0.0)
    ha_o[...] = ha
    hb_o[...] = hb
    oa[...] = jnp.dot(ha, wla[...], preferred_element_type=jnp.float32)
    ob[...] = jnp.dot(hb, wlb[...], preferred_element_type=jnp.float32)


_row_spec = pl.BlockSpec((BR, D), lambda i: (i, 0))
_w_spec = pl.BlockSpec((D, D), lambda i: (0, 0))
_pair_spec = pl.BlockSpec((NC, BR, D), lambda i: (0, i, 0))
_cnt_spec = pl.BlockSpec((NC, BR, 1), lambda i: (0, i, 0))

_lin1 = pl.pallas_call(
    _lin1_body,
    grid=(G,),
    in_specs=[_row_spec, _row_spec, _w_spec, _w_spec, _w_spec, _w_spec],
    out_specs=[_pair_spec, _row_spec, _row_spec],
    out_shape=[jax.ShapeDtypeStruct((NC, N, D), jnp.float32),
               jax.ShapeDtypeStruct((N, D), jnp.float32),
               jax.ShapeDtypeStruct((N, D), jnp.float32)],
)

_comb = pl.pallas_call(
    _comb_body,
    grid=(G,),
    in_specs=[_pair_spec, _cnt_spec, _row_spec, _row_spec,
              _w_spec, _w_spec, _w_spec, _w_spec],
    out_specs=[_pair_spec, _row_spec, _row_spec],
    out_shape=[jax.ShapeDtypeStruct((NC, N, D), jnp.float32),
               jax.ShapeDtypeStruct((N, D), jnp.float32),
               jax.ShapeDtypeStruct((N, D), jnp.float32)],
)

_fin = pl.pallas_call(
    _fin_body,
    grid=(G,),
    in_specs=[_pair_spec, _cnt_spec, _row_spec, _row_spec,
              pl.BlockSpec((D, 1), lambda i: (0, 0)),
              pl.BlockSpec((D, 1), lambda i: (0, 0))],
    out_specs=[_row_spec, _row_spec,
               pl.BlockSpec((BR, 1), lambda i: (i, 0)),
               pl.BlockSpec((BR, 1), lambda i: (i, 0))],
    out_shape=[jax.ShapeDtypeStruct((N, D), jnp.float32),
               jax.ShapeDtypeStruct((N, D), jnp.float32),
               jax.ShapeDtypeStruct((N, 1), jnp.float32),
               jax.ShapeDtypeStruct((N, 1), jnp.float32)],
)


def _prep_edges(edge_ab, edge_ba):
    npad = EP - E

    def one(edge, sbase):
        pad_src = sbase + lax.iota(jnp.int32, npad) % N
        pad_dst = N + (lax.iota(jnp.int32, npad) % 16)
        src = jnp.concatenate([edge[0] + sbase, pad_src])
        dst = jnp.concatenate([edge[1], pad_dst])
        return (src.reshape(NS, NCHUNK, CH), dst.reshape(NS, NCHUNK, CH))

    sab, dab = one(edge_ab, 0)
    sba, dba = one(edge_ba, N)
    return jnp.stack([sab, sba]), jnp.stack([dab, dba])


def kernel(x_a, x_b, edge_ab, edge_ba, W_src1_ab, W_tgt1_ab, W_src1_ba,
           W_tgt1_ba, W_src2_ab, W_tgt2_ab, W_src2_ba, W_tgt2_ba,
           W_lin_a, W_lin_b, b_lin_a, b_lin_b):
    srcr, dstr = _prep_edges(edge_ab, edge_ba)

    sxp1, tx1ab, tx1ba = _lin1(
        x_a, x_b, W_src1_ab, W_tgt1_ab, W_src1_ba, W_tgt1_ba)
    part1, cnt = _sc_scatter(sxp1.reshape(NC * N, D), srcr, dstr)
    cnt3 = cnt.reshape(NC, NPAD, 1)

    sxp2, tx2ab, tx2ba = _comb(part1, cnt3, tx1ab, tx1ba,
                               W_src2_ab, W_tgt2_ab, W_src2_ba, W_tgt2_ba)
    part2 = _sc_scatter_nc(sxp2.reshape(NC * N, D), srcr, dstr)

    ha, hb, oa, ob = _fin(part2, cnt3, tx2ab, tx2ba, W_lin_a, W_lin_b)
    return ha, hb, oa + b_lin_a, ob + b_lin_b
